# baseline (speedup 1.0000x reference)
import jax
import jax.numpy as jnp
from jax import lax
from jax.experimental import pallas as pl
from jax.experimental.pallas import tpu as pltpu
from jax.experimental.pallas import tpu_sc as plsc

NC = 2
NS = 16
L = 16
CH = 16
C = 256
HW = 4096
NCH = C // CH
BN = 32


def _vrsqrt(x):
    i = plsc.bitcast(x, jnp.int32)
    i = jnp.int32(0x5F3759DF) - lax.shift_right_logical(i, 1)
    y = plsc.bitcast(i, jnp.float32)
    for _ in range(4):
        y = y * (1.5 - 0.5 * x * y * y)
    return y


def _sc_body(v_hbm, m_hbm, g_hbm, o_hbm, mbuf, vb, sbuf, gbuf):
    bn = lax.axis_index("s") * NC + lax.axis_index("c")

    pltpu.sync_copy(m_hbm.at[bn], mbuf)
    pltpu.sync_copy(g_hbm, gbuf)
    gv = gbuf[pl.ds(0, L)]
    fgv = jnp.full((L,), gv[0], jnp.float32)
    pgv = jnp.full((L,), gv[1], jnp.float32)
    lane = jnp.arange(L, dtype=jnp.int32)

    def msum(h, acc):
        return acc + mbuf[pl.ds(h * L, L)]

    sm = jnp.sum(lax.fori_loop(0, HW // L, msum, jnp.zeros((L,), jnp.float32)))
    den_v = jnp.maximum(jnp.full((L,), sm, jnp.float32), 1e-6)
    fb_v = den_v <= 1e-5
    use_fb = sm <= 1e-5

    @pl.when(use_fb)
    def _():
        def fill(h, c):
            mbuf[pl.ds(h * L, L)] = jnp.ones((L,), jnp.float32)
            return c
        lax.fori_loop(0, HW // L, fill, 0)

    inv_den = jnp.where(fb_v, jnp.full((L,), 1.0 / HW, jnp.float32),
                        jnp.full((L,), 1.0, jnp.float32) / den_v)

    def p1_body(ch, carry):
        pltpu.sync_copy(v_hbm.at[bn, pl.ds(ch * CH, CH)], vb)

        def acc_body(h, accs):
            sl = pl.ds(h * L, L)
            m16 = mbuf[sl]
            return tuple(accs[r] + vb[r, sl] * m16 for r in range(CH))

        accs = lax.fori_loop(0, HW // L, acc_body,
                             tuple(jnp.zeros((L,), jnp.float32) for _ in range(CH)))
        sums = jnp.zeros((L,), jnp.float32)
        for r in range(CH):
            sums = jnp.where(lane == r, jnp.sum(accs[r]), sums)
        sbuf[pl.ds(ch * L, L)] = sums * inv_den
        return carry

    lax.fori_loop(0, NCH, p1_body, 0)

    def ss_body(g, acc):
        x = sbuf[pl.ds(g * L, L)]
        return acc + x * x

    ss = jnp.sum(lax.fori_loop(0, C // L, ss_body, jnp.zeros((L,), jnp.float32)))
    t1 = jnp.full((L,), ss, jnp.float32) + 1e-12
    r1 = _vrsqrt(t1)
    ss2 = jnp.full((L,), ss, jnp.float32) * r1 * r1
    r2 = _vrsqrt(ss2 + 1e-12)
    scale = r1 * r2 * pgv

    def scale_body(g, c):
        sl = pl.ds(g * L, L)
        sbuf[sl] = sbuf[sl] * scale
        return c

    lax.fori_loop(0, C // L, scale_body, 0)

    def p2_body(ch, carry):
        pltpu.sync_copy(v_hbm.at[bn, pl.ds(ch * CH, CH)], vb)
        qv = sbuf[pl.ds(ch * L, L)]
        qs = [jnp.full((L,), qv[r], jnp.float32) for r in range(CH)]

        def o_body(h, c):
            sl = pl.ds(h * L, L)
            for r in range(CH):
                vb[r, sl] = vb[r, sl] * fgv + qs[r]
            return c

        lax.fori_loop(0, HW // L, o_body, 0)
        pltpu.sync_copy(vb, o_hbm.at[bn, pl.ds(ch * CH, CH)])
        return carry

    lax.fori_loop(0, NCH, p2_body, 0)


def kernel(value_BNCHW, frame_feat_BCHW, mask_BNHW, proto_gate, frame_gate):
    B, N, Cd, H, W = value_BNCHW.shape
    v = value_BNCHW.reshape(BN, Cd, H * W)
    m = mask_BNHW.astype(jnp.float32).reshape(BN, H * W)
    g = jnp.zeros((L,), jnp.float32)
    g = g.at[0].set(jnp.asarray(frame_gate, jnp.float32))
    g = g.at[1].set(jnp.asarray(proto_gate, jnp.float32))

    mesh = plsc.VectorSubcoreMesh(core_axis_name="c", subcore_axis_name="s")
    out = pl.kernel(
        _sc_body,
        out_type=jax.ShapeDtypeStruct((BN, Cd, H * W), jnp.float32),
        mesh=mesh,
        compiler_params=pltpu.CompilerParams(needs_layout_passes=False),
        scratch_types=[
            pltpu.VMEM((HW,), jnp.float32),
            pltpu.VMEM((CH, HW), jnp.float32),
            pltpu.VMEM((C,), jnp.float32),
            pltpu.VMEM((L,), jnp.float32),
        ],
    )(v, m, g)
    return out.reshape(B, N, Cd, H, W)


# speedup vs baseline: 1.0726x; 1.0726x over previous
import jax
import jax.numpy as jnp
from jax import lax
from jax.experimental import pallas as pl
from jax.experimental.pallas import tpu as pltpu
from jax.experimental.pallas import tpu_sc as plsc

NC = 2
NS = 16
L = 16
CH = 8
C = 256
HW = 4096
NCH = C // CH
BN = 32


def _vrsqrt(x):
    i = plsc.bitcast(x, jnp.int32)
    i = jnp.int32(0x5F3759DF) - lax.shift_right_logical(i, 1)
    y = plsc.bitcast(i, jnp.float32)
    for _ in range(4):
        y = y * (1.5 - 0.5 * x * y * y)
    return y


def _sc_body(v_hbm, m_hbm, g_hbm, o_hbm, mbuf, vb0, vb1, sbuf, gbuf,
             si0, si1, so0, so1):
    bn = lax.axis_index("s") * NC + lax.axis_index("c")

    pltpu.sync_copy(m_hbm.at[bn], mbuf)
    pltpu.sync_copy(g_hbm, gbuf)
    gv = gbuf[pl.ds(0, L)]
    fgv = jnp.full((L,), gv[0], jnp.float32)
    pgv = jnp.full((L,), gv[1], jnp.float32)
    lane = jnp.arange(L, dtype=jnp.int32)

    def _in(ch, buf, sem):
        pltpu.async_copy(v_hbm.at[bn, pl.ds(ch * CH, CH)], buf, sem)

    def _win(buf, sem):
        pltpu.make_async_copy(v_hbm.at[bn, pl.ds(0, CH)], buf, sem).wait()

    def _out(ch, buf, sem):
        pltpu.async_copy(buf, o_hbm.at[bn, pl.ds(ch * CH, CH)], sem)

    def _wout(buf, sem):
        pltpu.make_async_copy(buf, o_hbm.at[bn, pl.ds(0, CH)], sem).wait()

    def msum(h, acc):
        return acc + mbuf[pl.ds(h * L, L)]

    sm = jnp.sum(lax.fori_loop(0, HW // L, msum, jnp.zeros((L,), jnp.float32)))
    den_v = jnp.maximum(jnp.full((L,), sm, jnp.float32), 1e-6)
    fb_v = den_v <= 1e-5
    use_fb = sm <= 1e-5

    @pl.when(use_fb)
    def _():
        def fill(h, c):
            mbuf[pl.ds(h * L, L)] = jnp.ones((L,), jnp.float32)
            return c
        lax.fori_loop(0, HW // L, fill, 0)

    inv_den = jnp.where(fb_v, jnp.full((L,), 1.0 / HW, jnp.float32),
                        jnp.full((L,), 1.0, jnp.float32) / den_v)

    def _accumulate(buf):
        def acc_body(h, accs):
            sl = pl.ds(h * L, L)
            m16 = mbuf[sl]
            return tuple(accs[r] + buf[r, sl] * m16 for r in range(CH))

        return lax.fori_loop(0, HW // L, acc_body,
                             tuple(jnp.zeros((L,), jnp.float32) for _ in range(CH)))

    _in(0, vb0, si0)

    def p1_pair(jp, c):
        ch0 = jp * 2
        _in(ch0 + 1, vb1, si1)
        _win(vb0, si0)
        a0 = _accumulate(vb0)

        @pl.when(ch0 + 2 < NCH)
        def _():
            _in(ch0 + 2, vb0, si0)

        _win(vb1, si1)
        a1 = _accumulate(vb1)

        @pl.when(ch0 + 3 < NCH)
        def _():
            _in(ch0 + 3, vb1, si1)

        sums = jnp.zeros((L,), jnp.float32)
        for r in range(CH):
            sums = jnp.where(lane == r, jnp.sum(a0[r]), sums)
            sums = jnp.where(lane == CH + r, jnp.sum(a1[r]), sums)
        sbuf[pl.ds(ch0 * CH, L)] = sums * inv_den
        return c

    lax.fori_loop(0, NCH // 2, p1_pair, 0)

    def ss_body(g, acc):
        x = sbuf[pl.ds(g * L, L)]
        return acc + x * x

    ss = jnp.sum(lax.fori_loop(0, C // L, ss_body, jnp.zeros((L,), jnp.float32)))
    t1 = jnp.full((L,), ss, jnp.float32) + 1e-12
    r1 = _vrsqrt(t1)
    ss2 = jnp.full((L,), ss, jnp.float32) * r1 * r1
    r2 = _vrsqrt(ss2 + 1e-12)
    scale = r1 * r2 * pgv

    def scale_body(g, c):
        sl = pl.ds(g * L, L)
        sbuf[sl] = sbuf[sl] * scale
        return c

    lax.fori_loop(0, C // L, scale_body, 0)

    def _compute_out(ch, buf):
        qv = sbuf[pl.ds(ch * CH, L)]
        qs = [jnp.full((L,), qv[r], jnp.float32) for r in range(CH)]

        def o_body(h, c):
            sl = pl.ds(h * L, L)
            for r in range(CH):
                buf[r, sl] = buf[r, sl] * fgv + qs[r]
            return c

        lax.fori_loop(0, HW // L, o_body, 0)

    _in(0, vb0, si0)

    def p2_pair(jp, c):
        ch0 = jp * 2
        _in(ch0 + 1, vb1, si1)
        _win(vb0, si0)
        _compute_out(ch0, vb0)
        _out(ch0, vb0, so0)
        _win(vb1, si1)
        _compute_out(ch0 + 1, vb1)
        _out(ch0 + 1, vb1, so1)

        @pl.when(ch0 + 2 < NCH)
        def _():
            _wout(vb0, so0)
            _in(ch0 + 2, vb0, si0)
            _wout(vb1, so1)
            _in(ch0 + 3, vb1, si1)

        return c

    lax.fori_loop(0, NCH // 2, p2_pair, 0)
    _wout(vb0, so0)
    _wout(vb1, so1)


def kernel(value_BNCHW, frame_feat_BCHW, mask_BNHW, proto_gate, frame_gate):
    B, N, Cd, H, W = value_BNCHW.shape
    v = value_BNCHW.reshape(BN, Cd, H * W)
    m = mask_BNHW.astype(jnp.float32).reshape(BN, H * W)
    g = jnp.zeros((L,), jnp.float32)
    g = g.at[0].set(jnp.asarray(frame_gate, jnp.float32))
    g = g.at[1].set(jnp.asarray(proto_gate, jnp.float32))

    mesh = plsc.VectorSubcoreMesh(core_axis_name="c", subcore_axis_name="s")
    out = pl.kernel(
        _sc_body,
        out_type=jax.ShapeDtypeStruct((BN, Cd, H * W), jnp.float32),
        mesh=mesh,
        compiler_params=pltpu.CompilerParams(needs_layout_passes=False),
        scratch_types=[
            pltpu.VMEM((HW,), jnp.float32),
            pltpu.VMEM((CH, HW), jnp.float32),
            pltpu.VMEM((CH, HW), jnp.float32),
            pltpu.VMEM((C + L,), jnp.float32),
            pltpu.VMEM((L,), jnp.float32),
            pltpu.SemaphoreType.DMA,
            pltpu.SemaphoreType.DMA,
            pltpu.SemaphoreType.DMA,
            pltpu.SemaphoreType.DMA,
        ],
    )(v, m, g)
    return out.reshape(B, N, Cd, H, W)
